# Initial kernel scaffold; baseline (speedup 1.0000x reference)
import jax
import jax.numpy as jnp
from jax.experimental import pallas as pl


def kernel(x, edge_index, lp_edges, PI, W1, b1, W2, b2, Wl1, bl1, Wl, bl):
    raise NotImplementedError("write your pallas kernel here")


# speedup vs baseline: 13.7559x; 13.7559x over previous
import functools

import jax
import jax.numpy as jnp
from jax import lax
from jax.experimental import pallas as pl
from jax.experimental.pallas import tpu as pltpu
from jax.experimental.pallas import tpu_sc as plsc

N_NODES = 10000
N_EDGES = 320000
F_IN = 128
H1 = 100
H1P = 112
H2 = 16
E_LP = 100000
PI_D = 25

NC = 2
NS = 16
NW = NC * NS
CHUNK = 128

EDGE_PW = N_EDGES // NW
KC_EDGE = -(-EDGE_PW // CHUNK)
EDGE_PW_PAD = KC_EDGE * CHUNK

LP_PW = E_LP // NW
KC_LP = -(-LP_PW // CHUNK)
LP_PW_PAD = KC_LP * CHUNK
E_LP_PAD = LP_PW_PAD * NW

NPAD = 10112
ROWS_PT = NPAD // NS

ROW_BLK = 2000
LP_BLK = 5000


def _sc_mesh():
    return plsc.VectorSubcoreMesh(core_axis_name="c", subcore_axis_name="s")


@functools.partial(
    pl.kernel,
    out_type=jax.ShapeDtypeStruct((NC, NPAD, H2), jnp.float32),
    mesh=_sc_mesh(),
    compiler_params=pltpu.CompilerParams(use_tc_tiling_on_sc=False),
    scratch_types=[
        pltpu.VMEM((KC_EDGE, CHUNK), jnp.int32),
        pltpu.VMEM((CHUNK, H2), jnp.float32),
        pltpu.VMEM_SHARED((NPAD, H2), jnp.float32),
    ],
)
def _sc_degree(dst_hbm, ones_hbm, zeros_hbm, out_hbm, dst_v, ones_v, acc):
    c = lax.axis_index("c")
    s = lax.axis_index("s")
    w = s * NC + c
    pltpu.sync_copy(zeros_hbm, acc.at[pl.ds(s * ROWS_PT, ROWS_PT)])
    pltpu.sync_copy(dst_hbm.at[w], dst_v)
    pltpu.sync_copy(ones_hbm, ones_v)
    plsc.subcore_barrier()

    def body(j, carry):
        pltpu.sync_copy(ones_v, acc.at[dst_v.at[j]], add=True)
        return carry

    lax.fori_loop(0, KC_EDGE, body, 0)
    plsc.subcore_barrier()
    pltpu.sync_copy(acc.at[pl.ds(s * ROWS_PT, ROWS_PT)],
                    out_hbm.at[c, pl.ds(s * ROWS_PT, ROWS_PT)])


def _make_sc_agg(d):
    @functools.partial(
        pl.kernel,
        out_type=jax.ShapeDtypeStruct((NC, NPAD, d), jnp.float32),
        mesh=_sc_mesh(),
        compiler_params=pltpu.CompilerParams(use_tc_tiling_on_sc=False),
        scratch_types=[
            pltpu.VMEM((KC_EDGE, CHUNK), jnp.int32),
            pltpu.VMEM((KC_EDGE, CHUNK), jnp.int32),
            pltpu.VMEM((CHUNK, d), jnp.float32),
            pltpu.VMEM_SHARED((NPAD, d), jnp.float32),
            pltpu.SemaphoreType.DMA,
        ],
    )
    def agg(tbl_hbm, src_hbm, dst_hbm, zeros_hbm, out_hbm,
            src_v, dst_v, buf, acc, sem):
        c = lax.axis_index("c")
        s = lax.axis_index("s")
        w = s * NC + c
        pltpu.sync_copy(zeros_hbm, acc.at[pl.ds(s * ROWS_PT, ROWS_PT)])
        pltpu.sync_copy(src_hbm.at[w], src_v)
        pltpu.sync_copy(dst_hbm.at[w], dst_v)
        plsc.subcore_barrier()

        def body(j, carry):
            pltpu.async_copy(tbl_hbm.at[src_v.at[j]], buf, sem).wait()
            pltpu.sync_copy(buf, acc.at[dst_v.at[j]], add=True)
            return carry

        lax.fori_loop(0, KC_EDGE, body, 0)
        plsc.subcore_barrier()
        pltpu.sync_copy(acc.at[pl.ds(s * ROWS_PT, ROWS_PT)],
                        out_hbm.at[c, pl.ds(s * ROWS_PT, ROWS_PT)])

    return agg


_sc_agg_h1 = _make_sc_agg(H1P)
_sc_agg_h2 = _make_sc_agg(H2)


@functools.partial(
    pl.kernel,
    out_type=jax.ShapeDtypeStruct((E_LP_PAD, H2), jnp.float32),
    mesh=_sc_mesh(),
    compiler_params=pltpu.CompilerParams(use_tc_tiling_on_sc=False),
    scratch_types=[
        pltpu.VMEM((KC_LP, CHUNK), jnp.int32),
        pltpu.VMEM((KC_LP, CHUNK), jnp.int32),
        pltpu.VMEM((CHUNK, H2), jnp.float32),
        pltpu.VMEM((CHUNK, H2), jnp.float32),
        pltpu.VMEM((CHUNK, H2), jnp.float32),
        pltpu.SemaphoreType.DMA,
        pltpu.SemaphoreType.DMA,
    ],
)
def _sc_lp_sqdist(emb_hbm, a_hbm, b_hbm, out_hbm,
                  a_v, b_v, bufa, bufb, bufd, sema, semb):
    c = lax.axis_index("c")
    s = lax.axis_index("s")
    w = s * NC + c
    base = w * LP_PW_PAD
    pltpu.sync_copy(a_hbm.at[w], a_v)
    pltpu.sync_copy(b_hbm.at[w], b_v)

    def body(j, carry):
        cpa = pltpu.async_copy(emb_hbm.at[a_v.at[j]], bufa, sema)
        cpb = pltpu.async_copy(emb_hbm.at[b_v.at[j]], bufb, semb)
        cpa.wait()
        cpb.wait()

        def row(r, cc):
            dv = bufa[r] - bufb[r]
            bufd[r] = dv * dv
            return cc

        lax.fori_loop(0, CHUNK, row, 0)
        pltpu.sync_copy(bufd, out_hbm.at[pl.ds(base + j * CHUNK, CHUNK)])
        return carry

    lax.fori_loop(0, KC_LP, body, 0)


def _deg_dinv(parts):
    deg = parts[0, :, 0] + parts[1, :, 0] + 1.0
    return lax.rsqrt(jnp.maximum(deg, 1.0))


def _tc_encode1_body(degp_ref, x_ref, w1_ref, o_ref):
    dinv = _deg_dinv(degp_ref[...])
    hw = jnp.dot(x_ref[...], w1_ref[...], preferred_element_type=jnp.float32)
    o_ref[...] = hw * dinv[:, None]


def _tc_encode2_body(degp_ref, accp_ref, g1_ref, w2_ref, b1_ref, o_ref):
    dinv = _deg_dinv(degp_ref[...])
    a = accp_ref[0] + accp_ref[1] + g1_ref[...]
    h = jnp.maximum(a * dinv[:, None] + b1_ref[...], 0.0)
    o_ref[...] = jnp.dot(h, w2_ref[...],
                         preferred_element_type=jnp.float32) * dinv[:, None]


def _tc_emb_body(degp_ref, accp_ref, g2_ref, b2_ref, o_ref):
    dinv = _deg_dinv(degp_ref[...])
    a = accp_ref[0] + accp_ref[1] + g2_ref[...]
    emb = jnp.maximum(a * dinv[:, None] + b2_ref[...], 0.0)
    nrm = jnp.sqrt(jnp.sum(emb * emb, axis=1, keepdims=True))
    scale = jnp.where(nrm > 1.0, 1.0 / (nrm + 1e-7), 1.0)
    o_ref[...] = emb * scale


def _tc_decode_body(sq_ref, pi_ref, wa_ref, wb_ref, bl1_ref, wl_ref,
                    bl_ref, o_ref):
    z = (jnp.dot(sq_ref[...], wa_ref[...], preferred_element_type=jnp.float32)
         + jnp.dot(pi_ref[...], wb_ref[...], preferred_element_type=jnp.float32)
         + bl1_ref[...])
    z = jnp.where(z > 0, z, 0.2 * z)
    sv = jnp.dot(z, wl_ref[...], preferred_element_type=jnp.float32)[:, 0]
    s = jnp.clip(jnp.abs(sv + bl_ref[0, 0]), 0.0, 40.0)
    o_ref[...] = (1.0 / (jnp.exp(s - 2.0) + 1.0))[:, None]


def kernel(x, edge_index, lp_edges, PI, W1, b1, W2, b2, Wl1, bl1, Wl, bl):
    f32 = jnp.float32
    src = edge_index[0]
    dst = edge_index[1]
    pad = EDGE_PW_PAD - EDGE_PW
    srcm = jnp.pad(src.reshape(NW, EDGE_PW), ((0, 0), (0, pad))
                   ).reshape(NW, KC_EDGE, CHUNK)
    dstm = jnp.pad(dst.reshape(NW, EDGE_PW), ((0, 0), (0, pad)),
                   constant_values=N_NODES).reshape(NW, KC_EDGE, CHUNK)
    lpa = jnp.pad(lp_edges[:, 0], (0, E_LP_PAD - E_LP)
                  ).reshape(NW, KC_LP, CHUNK)
    lpb = jnp.pad(lp_edges[:, 1], (0, E_LP_PAD - E_LP)
                  ).reshape(NW, KC_LP, CHUNK)

    ones_blk = jnp.ones((CHUNK, H2), f32)
    zeros_h2 = jnp.zeros((ROWS_PT, H2), f32)
    zeros_h1 = jnp.zeros((ROWS_PT, H1P), f32)

    W1p = jnp.pad(W1, ((0, 0), (0, H1P - H1)))
    W2p = jnp.pad(W2, ((0, H1P - H1), (0, 0)))
    b1p = jnp.pad(b1, (0, H1P - H1)).reshape(1, H1P)

    deg_parts = _sc_degree(dstm, ones_blk, zeros_h2)
    degp = deg_parts[:, :N_NODES, :]

    g1 = pl.pallas_call(
        _tc_encode1_body,
        grid=(N_NODES // ROW_BLK,),
        in_specs=[
            pl.BlockSpec((2, ROW_BLK, H2), lambda i: (0, i, 0)),
            pl.BlockSpec((ROW_BLK, F_IN), lambda i: (i, 0)),
            pl.BlockSpec((F_IN, H1P), lambda i: (0, 0)),
        ],
        out_specs=pl.BlockSpec((ROW_BLK, H1P), lambda i: (i, 0)),
        out_shape=jax.ShapeDtypeStruct((N_NODES, H1P), f32),
    )(degp, x, W1p)

    acc1 = _sc_agg_h1(g1, srcm, dstm, zeros_h1)

    g2 = pl.pallas_call(
        _tc_encode2_body,
        grid=(N_NODES // ROW_BLK,),
        in_specs=[
            pl.BlockSpec((2, ROW_BLK, H2), lambda i: (0, i, 0)),
            pl.BlockSpec((2, ROW_BLK, H1P), lambda i: (0, i, 0)),
            pl.BlockSpec((ROW_BLK, H1P), lambda i: (i, 0)),
            pl.BlockSpec((H1P, H2), lambda i: (0, 0)),
            pl.BlockSpec((1, H1P), lambda i: (0, 0)),
        ],
        out_specs=pl.BlockSpec((ROW_BLK, H2), lambda i: (i, 0)),
        out_shape=jax.ShapeDtypeStruct((N_NODES, H2), f32),
    )(degp, acc1[:, :N_NODES, :], g1, W2p, b1p)

    acc2 = _sc_agg_h2(g2, srcm, dstm, zeros_h2)

    emb = pl.pallas_call(
        _tc_emb_body,
        grid=(N_NODES // ROW_BLK,),
        in_specs=[
            pl.BlockSpec((2, ROW_BLK, H2), lambda i: (0, i, 0)),
            pl.BlockSpec((2, ROW_BLK, H2), lambda i: (0, i, 0)),
            pl.BlockSpec((ROW_BLK, H2), lambda i: (i, 0)),
            pl.BlockSpec((1, H2), lambda i: (0, 0)),
        ],
        out_specs=pl.BlockSpec((ROW_BLK, H2), lambda i: (i, 0)),
        out_shape=jax.ShapeDtypeStruct((N_NODES, H2), f32),
    )(degp, acc2[:, :N_NODES, :], g2, b2.reshape(1, H2))

    sq = _sc_lp_sqdist(emb, lpa, lpb)

    prob = pl.pallas_call(
        _tc_decode_body,
        grid=(E_LP // LP_BLK,),
        in_specs=[
            pl.BlockSpec((LP_BLK, H2), lambda i: (i, 0)),
            pl.BlockSpec((LP_BLK, PI_D), lambda i: (i, 0)),
            pl.BlockSpec((H2, PI_D), lambda i: (0, 0)),
            pl.BlockSpec((PI_D, PI_D), lambda i: (0, 0)),
            pl.BlockSpec((1, PI_D), lambda i: (0, 0)),
            pl.BlockSpec((PI_D, 1), lambda i: (0, 0)),
            pl.BlockSpec((1, 1), lambda i: (0, 0)),
        ],
        out_specs=pl.BlockSpec((LP_BLK, 1), lambda i: (i, 0)),
        out_shape=jax.ShapeDtypeStruct((E_LP, 1), f32),
    )(sq[:E_LP], PI, Wl1[:H2], Wl1[H2:], bl1.reshape(1, PI_D), Wl,
      bl.reshape(1, 1))
    return prob.reshape(-1)


# speedup vs baseline: 15.9247x; 1.1577x over previous
import functools

import jax
import jax.numpy as jnp
from jax import lax
from jax.experimental import pallas as pl
from jax.experimental.pallas import tpu as pltpu
from jax.experimental.pallas import tpu_sc as plsc

N_NODES = 10000
N_EDGES = 320000
F_IN = 128
H1 = 100
H1P = 112
H2 = 16
E_LP = 100000
PI_D = 25

NC = 2
NS = 16
NW = NC * NS
CHUNK = 128

EDGE_PW = N_EDGES // NW
KC_EDGE = -(-EDGE_PW // CHUNK)
EDGE_PW_PAD = KC_EDGE * CHUNK

LP_PW = E_LP // NW
KC_LP = -(-LP_PW // CHUNK)
LP_PW_PAD = KC_LP * CHUNK
E_LP_PAD = LP_PW_PAD * NW

NPAD = 10112
ROWS_PT = NPAD // NS

ROW_BLK = 2000
LP_BLK = 5000


def _sc_mesh():
    return plsc.VectorSubcoreMesh(core_axis_name="c", subcore_axis_name="s")


@functools.partial(
    pl.kernel,
    out_type=jax.ShapeDtypeStruct((NC, NPAD, H2), jnp.float32),
    mesh=_sc_mesh(),
    compiler_params=pltpu.CompilerParams(use_tc_tiling_on_sc=False),
    scratch_types=[
        pltpu.VMEM((KC_EDGE, CHUNK), jnp.int32),
        pltpu.VMEM((CHUNK, H2), jnp.float32),
        pltpu.VMEM_SHARED((NPAD, H2), jnp.float32),
    ],
)
def _sc_degree(dst_hbm, ones_hbm, zeros_hbm, out_hbm, dst_v, ones_v, acc):
    c = lax.axis_index("c")
    s = lax.axis_index("s")
    w = s * NC + c
    pltpu.sync_copy(zeros_hbm, acc.at[pl.ds(s * ROWS_PT, ROWS_PT)])
    pltpu.sync_copy(dst_hbm.at[w], dst_v)
    pltpu.sync_copy(ones_hbm, ones_v)
    plsc.subcore_barrier()

    def body(j, carry):
        pltpu.sync_copy(ones_v, acc.at[dst_v.at[j]], add=True)
        return carry

    lax.fori_loop(0, KC_EDGE, body, 0)
    plsc.subcore_barrier()
    pltpu.sync_copy(acc.at[pl.ds(s * ROWS_PT, ROWS_PT)],
                    out_hbm.at[c, pl.ds(s * ROWS_PT, ROWS_PT)])


def _make_sc_agg(d):
    @functools.partial(
        pl.kernel,
        out_type=jax.ShapeDtypeStruct((NC, NPAD, d), jnp.float32),
        mesh=_sc_mesh(),
        compiler_params=pltpu.CompilerParams(use_tc_tiling_on_sc=False),
        scratch_types=[
            pltpu.VMEM((KC_EDGE, CHUNK), jnp.int32),
            pltpu.VMEM((KC_EDGE, CHUNK), jnp.int32),
            pltpu.VMEM((CHUNK, d), jnp.float32),
            pltpu.VMEM_SHARED((NPAD, d), jnp.float32),
            pltpu.SemaphoreType.DMA,
        ],
    )
    def agg(tbl_hbm, src_hbm, dst_hbm, zeros_hbm, out_hbm,
            src_v, dst_v, buf, acc, sem):
        c = lax.axis_index("c")
        s = lax.axis_index("s")
        w = s * NC + c
        pltpu.sync_copy(zeros_hbm, acc.at[pl.ds(s * ROWS_PT, ROWS_PT)])
        pltpu.sync_copy(src_hbm.at[w], src_v)
        pltpu.sync_copy(dst_hbm.at[w], dst_v)
        plsc.subcore_barrier()

        def body(j, carry):
            pltpu.async_copy(tbl_hbm.at[src_v.at[j]], buf, sem).wait()
            pltpu.sync_copy(buf, acc.at[dst_v.at[j]], add=True)
            return carry

        lax.fori_loop(0, KC_EDGE, body, 0)
        plsc.subcore_barrier()
        pltpu.sync_copy(acc.at[pl.ds(s * ROWS_PT, ROWS_PT)],
                        out_hbm.at[c, pl.ds(s * ROWS_PT, ROWS_PT)])

    return agg


_sc_agg_h1 = _make_sc_agg(H1P)


NROWS_STAGE = N_NODES // NS


@functools.partial(
    pl.kernel,
    out_type=jax.ShapeDtypeStruct((NC, NPAD, H2), jnp.float32),
    mesh=_sc_mesh(),
    compiler_params=pltpu.CompilerParams(use_tc_tiling_on_sc=False),
    scratch_types=[
        pltpu.VMEM((KC_EDGE, CHUNK), jnp.int32),
        pltpu.VMEM((KC_EDGE, CHUNK), jnp.int32),
        pltpu.VMEM((CHUNK, H2), jnp.float32),
        pltpu.VMEM_SHARED((N_NODES, H2), jnp.float32),
        pltpu.VMEM_SHARED((NPAD, H2), jnp.float32),
        pltpu.SemaphoreType.DMA,
    ],
)
def _sc_agg_h2(tbl_hbm, src_hbm, dst_hbm, zeros_hbm, out_hbm,
               src_v, dst_v, buf, tbl_s, acc, sem):
    c = lax.axis_index("c")
    s = lax.axis_index("s")
    w = s * NC + c
    pltpu.sync_copy(zeros_hbm, acc.at[pl.ds(s * ROWS_PT, ROWS_PT)])
    pltpu.sync_copy(tbl_hbm.at[pl.ds(s * NROWS_STAGE, NROWS_STAGE)],
                    tbl_s.at[pl.ds(s * NROWS_STAGE, NROWS_STAGE)])
    pltpu.sync_copy(src_hbm.at[w], src_v)
    pltpu.sync_copy(dst_hbm.at[w], dst_v)
    plsc.subcore_barrier()

    def body(j, carry):
        pltpu.async_copy(tbl_s.at[src_v.at[j]], buf, sem).wait()
        pltpu.sync_copy(buf, acc.at[dst_v.at[j]], add=True)
        return carry

    lax.fori_loop(0, KC_EDGE, body, 0)
    plsc.subcore_barrier()
    pltpu.sync_copy(acc.at[pl.ds(s * ROWS_PT, ROWS_PT)],
                    out_hbm.at[c, pl.ds(s * ROWS_PT, ROWS_PT)])


@functools.partial(
    pl.kernel,
    out_type=jax.ShapeDtypeStruct((E_LP_PAD, H2), jnp.float32),
    mesh=_sc_mesh(),
    compiler_params=pltpu.CompilerParams(use_tc_tiling_on_sc=False),
    scratch_types=[
        pltpu.VMEM((KC_LP, CHUNK), jnp.int32),
        pltpu.VMEM((KC_LP, CHUNK), jnp.int32),
        pltpu.VMEM((CHUNK, H2), jnp.float32),
        pltpu.VMEM((CHUNK, H2), jnp.float32),
        pltpu.VMEM((CHUNK, H2), jnp.float32),
        pltpu.VMEM_SHARED((N_NODES, H2), jnp.float32),
        pltpu.SemaphoreType.DMA,
        pltpu.SemaphoreType.DMA,
    ],
)
def _sc_lp_sqdist(emb_hbm, a_hbm, b_hbm, out_hbm,
                  a_v, b_v, bufa, bufb, bufd, emb_s, sema, semb):
    c = lax.axis_index("c")
    s = lax.axis_index("s")
    w = s * NC + c
    base = w * LP_PW_PAD
    pltpu.sync_copy(emb_hbm.at[pl.ds(s * NROWS_STAGE, NROWS_STAGE)],
                    emb_s.at[pl.ds(s * NROWS_STAGE, NROWS_STAGE)])
    pltpu.sync_copy(a_hbm.at[w], a_v)
    pltpu.sync_copy(b_hbm.at[w], b_v)
    plsc.subcore_barrier()

    def body(j, carry):
        cpa = pltpu.async_copy(emb_s.at[a_v.at[j]], bufa, sema)
        cpb = pltpu.async_copy(emb_s.at[b_v.at[j]], bufb, semb)
        cpa.wait()
        cpb.wait()

        def row(r, cc):
            dv = bufa[r] - bufb[r]
            bufd[r] = dv * dv
            return cc

        lax.fori_loop(0, CHUNK, row, 0)
        pltpu.sync_copy(bufd, out_hbm.at[pl.ds(base + j * CHUNK, CHUNK)])
        return carry

    lax.fori_loop(0, KC_LP, body, 0)


def _deg_dinv(parts):
    deg = parts[0, :, 0] + parts[1, :, 0] + 1.0
    return lax.rsqrt(jnp.maximum(deg, 1.0))


def _tc_encode1_body(degp_ref, x_ref, w1_ref, o_ref):
    dinv = _deg_dinv(degp_ref[...])
    hw = jnp.dot(x_ref[...], w1_ref[...], preferred_element_type=jnp.float32)
    o_ref[...] = hw * dinv[:, None]


def _tc_encode2_body(degp_ref, accp_ref, g1_ref, w2_ref, b1_ref, o_ref):
    dinv = _deg_dinv(degp_ref[...])
    a = accp_ref[0] + accp_ref[1] + g1_ref[...]
    h = jnp.maximum(a * dinv[:, None] + b1_ref[...], 0.0)
    o_ref[...] = jnp.dot(h, w2_ref[...],
                         preferred_element_type=jnp.float32) * dinv[:, None]


def _tc_emb_body(degp_ref, accp_ref, g2_ref, b2_ref, o_ref):
    dinv = _deg_dinv(degp_ref[...])
    a = accp_ref[0] + accp_ref[1] + g2_ref[...]
    emb = jnp.maximum(a * dinv[:, None] + b2_ref[...], 0.0)
    nrm = jnp.sqrt(jnp.sum(emb * emb, axis=1, keepdims=True))
    scale = jnp.where(nrm > 1.0, 1.0 / (nrm + 1e-7), 1.0)
    o_ref[...] = emb * scale


def _tc_decode_body(sq_ref, pi_ref, wa_ref, wb_ref, bl1_ref, wl_ref,
                    bl_ref, o_ref):
    z = (jnp.dot(sq_ref[...], wa_ref[...], preferred_element_type=jnp.float32)
         + jnp.dot(pi_ref[...], wb_ref[...], preferred_element_type=jnp.float32)
         + bl1_ref[...])
    z = jnp.where(z > 0, z, 0.2 * z)
    sv = jnp.dot(z, wl_ref[...], preferred_element_type=jnp.float32)[:, 0]
    s = jnp.clip(jnp.abs(sv + bl_ref[0, 0]), 0.0, 40.0)
    o_ref[...] = (1.0 / (jnp.exp(s - 2.0) + 1.0))[:, None]


def kernel(x, edge_index, lp_edges, PI, W1, b1, W2, b2, Wl1, bl1, Wl, bl):
    f32 = jnp.float32
    src = edge_index[0]
    dst = edge_index[1]
    pad = EDGE_PW_PAD - EDGE_PW
    srcm = jnp.pad(src.reshape(NW, EDGE_PW), ((0, 0), (0, pad))
                   ).reshape(NW, KC_EDGE, CHUNK)
    dstm = jnp.pad(dst.reshape(NW, EDGE_PW), ((0, 0), (0, pad)),
                   constant_values=N_NODES).reshape(NW, KC_EDGE, CHUNK)
    lpa = jnp.pad(lp_edges[:, 0], (0, E_LP_PAD - E_LP)
                  ).reshape(NW, KC_LP, CHUNK)
    lpb = jnp.pad(lp_edges[:, 1], (0, E_LP_PAD - E_LP)
                  ).reshape(NW, KC_LP, CHUNK)

    ones_blk = jnp.ones((CHUNK, H2), f32)
    zeros_h2 = jnp.zeros((ROWS_PT, H2), f32)
    zeros_h1 = jnp.zeros((ROWS_PT, H1P), f32)

    W1p = jnp.pad(W1, ((0, 0), (0, H1P - H1)))
    W2p = jnp.pad(W2, ((0, H1P - H1), (0, 0)))
    b1p = jnp.pad(b1, (0, H1P - H1)).reshape(1, H1P)

    deg_parts = _sc_degree(dstm, ones_blk, zeros_h2)
    degp = deg_parts[:, :N_NODES, :]

    g1 = pl.pallas_call(
        _tc_encode1_body,
        grid=(N_NODES // ROW_BLK,),
        in_specs=[
            pl.BlockSpec((2, ROW_BLK, H2), lambda i: (0, i, 0)),
            pl.BlockSpec((ROW_BLK, F_IN), lambda i: (i, 0)),
            pl.BlockSpec((F_IN, H1P), lambda i: (0, 0)),
        ],
        out_specs=pl.BlockSpec((ROW_BLK, H1P), lambda i: (i, 0)),
        out_shape=jax.ShapeDtypeStruct((N_NODES, H1P), f32),
    )(degp, x, W1p)

    acc1 = _sc_agg_h1(g1, srcm, dstm, zeros_h1)

    g2 = pl.pallas_call(
        _tc_encode2_body,
        grid=(N_NODES // ROW_BLK,),
        in_specs=[
            pl.BlockSpec((2, ROW_BLK, H2), lambda i: (0, i, 0)),
            pl.BlockSpec((2, ROW_BLK, H1P), lambda i: (0, i, 0)),
            pl.BlockSpec((ROW_BLK, H1P), lambda i: (i, 0)),
            pl.BlockSpec((H1P, H2), lambda i: (0, 0)),
            pl.BlockSpec((1, H1P), lambda i: (0, 0)),
        ],
        out_specs=pl.BlockSpec((ROW_BLK, H2), lambda i: (i, 0)),
        out_shape=jax.ShapeDtypeStruct((N_NODES, H2), f32),
    )(degp, acc1[:, :N_NODES, :], g1, W2p, b1p)

    acc2 = _sc_agg_h2(g2, srcm, dstm, zeros_h2)

    emb = pl.pallas_call(
        _tc_emb_body,
        grid=(N_NODES // ROW_BLK,),
        in_specs=[
            pl.BlockSpec((2, ROW_BLK, H2), lambda i: (0, i, 0)),
            pl.BlockSpec((2, ROW_BLK, H2), lambda i: (0, i, 0)),
            pl.BlockSpec((ROW_BLK, H2), lambda i: (i, 0)),
            pl.BlockSpec((1, H2), lambda i: (0, 0)),
        ],
        out_specs=pl.BlockSpec((ROW_BLK, H2), lambda i: (i, 0)),
        out_shape=jax.ShapeDtypeStruct((N_NODES, H2), f32),
    )(degp, acc2[:, :N_NODES, :], g2, b2.reshape(1, H2))

    sq = _sc_lp_sqdist(emb, lpa, lpb)

    prob = pl.pallas_call(
        _tc_decode_body,
        grid=(E_LP // LP_BLK,),
        in_specs=[
            pl.BlockSpec((LP_BLK, H2), lambda i: (i, 0)),
            pl.BlockSpec((LP_BLK, PI_D), lambda i: (i, 0)),
            pl.BlockSpec((H2, PI_D), lambda i: (0, 0)),
            pl.BlockSpec((PI_D, PI_D), lambda i: (0, 0)),
            pl.BlockSpec((1, PI_D), lambda i: (0, 0)),
            pl.BlockSpec((PI_D, 1), lambda i: (0, 0)),
            pl.BlockSpec((1, 1), lambda i: (0, 0)),
        ],
        out_specs=pl.BlockSpec((LP_BLK, 1), lambda i: (i, 0)),
        out_shape=jax.ShapeDtypeStruct((E_LP, 1), f32),
    )(sq[:E_LP], PI, Wl1[:H2], Wl1[H2:], bl1.reshape(1, PI_D), Wl,
      bl.reshape(1, 1))
    return prob.reshape(-1)


# speedup vs baseline: 17.2595x; 1.0838x over previous
import functools

import jax
import jax.numpy as jnp
from jax import lax
from jax.experimental import pallas as pl
from jax.experimental.pallas import tpu as pltpu
from jax.experimental.pallas import tpu_sc as plsc

N_NODES = 10000
N_EDGES = 320000
F_IN = 128
H1 = 100
H1P = 128
H1H = 64
H2 = 16
E_LP = 100000
PI_D = 25

NC = 2
NS = 16
NW = NC * NS
CHUNK = 128

EDGE_PW = N_EDGES // NW
KC_EDGE = -(-EDGE_PW // CHUNK)
EDGE_PW_PAD = KC_EDGE * CHUNK

LP_PW = E_LP // NW
KC_LP = -(-LP_PW // CHUNK)
LP_PW_PAD = KC_LP * CHUNK
E_LP_PAD = LP_PW_PAD * NW

NPAD = 10112
ROWS_PT = NPAD // NS

ROW_BLK = 2000
LP_BLK = 5000


def _sc_mesh():
    return plsc.VectorSubcoreMesh(core_axis_name="c", subcore_axis_name="s")


@functools.partial(
    pl.kernel,
    out_type=jax.ShapeDtypeStruct((NC, NPAD, H2), jnp.float32),
    mesh=_sc_mesh(),
    compiler_params=pltpu.CompilerParams(use_tc_tiling_on_sc=False),
    scratch_types=[
        pltpu.VMEM((KC_EDGE, CHUNK), jnp.int32),
        pltpu.VMEM((CHUNK, H2), jnp.float32),
        pltpu.VMEM_SHARED((NPAD, H2), jnp.float32),
    ],
)
def _sc_degree(dst_hbm, ones_hbm, zeros_hbm, out_hbm, dst_v, ones_v, acc):
    c = lax.axis_index("c")
    s = lax.axis_index("s")
    w = s * NC + c
    pltpu.sync_copy(zeros_hbm, acc.at[pl.ds(s * ROWS_PT, ROWS_PT)])
    pltpu.sync_copy(dst_hbm.at[w], dst_v)
    pltpu.sync_copy(ones_hbm, ones_v)
    plsc.subcore_barrier()

    def body(j, carry):
        pltpu.sync_copy(ones_v, acc.at[dst_v.at[j]], add=True)
        return carry

    lax.fori_loop(0, KC_EDGE, body, 0)
    plsc.subcore_barrier()
    pltpu.sync_copy(acc.at[pl.ds(s * ROWS_PT, ROWS_PT)],
                    out_hbm.at[c, pl.ds(s * ROWS_PT, ROWS_PT)])


NROWS_STAGE = N_NODES // NS


def _make_sc_agg_staged(d):
    @functools.partial(
        pl.kernel,
        out_type=jax.ShapeDtypeStruct((NC, NPAD, d), jnp.float32),
        mesh=_sc_mesh(),
        compiler_params=pltpu.CompilerParams(use_tc_tiling_on_sc=False),
        scratch_types=[
            pltpu.VMEM((KC_EDGE, CHUNK), jnp.int32),
            pltpu.VMEM((KC_EDGE, CHUNK), jnp.int32),
            pltpu.VMEM((CHUNK, d), jnp.float32),
            pltpu.VMEM_SHARED((N_NODES, d), jnp.float32),
            pltpu.VMEM_SHARED((NPAD, d), jnp.float32),
            pltpu.SemaphoreType.DMA,
        ],
    )
    def agg(tbl_hbm, src_hbm, dst_hbm, zeros_hbm, out_hbm,
            src_v, dst_v, buf, tbl_s, acc, sem):
        c = lax.axis_index("c")
        s = lax.axis_index("s")
        w = s * NC + c
        pltpu.sync_copy(zeros_hbm, acc.at[pl.ds(s * ROWS_PT, ROWS_PT)])
        pltpu.sync_copy(tbl_hbm.at[pl.ds(s * NROWS_STAGE, NROWS_STAGE)],
                        tbl_s.at[pl.ds(s * NROWS_STAGE, NROWS_STAGE)])
        pltpu.sync_copy(src_hbm.at[w], src_v)
        pltpu.sync_copy(dst_hbm.at[w], dst_v)
        plsc.subcore_barrier()

        def body(j, carry):
            pltpu.async_copy(tbl_s.at[src_v.at[j]], buf, sem).wait()
            pltpu.sync_copy(buf, acc.at[dst_v.at[j]], add=True)
            return carry

        lax.fori_loop(0, KC_EDGE, body, 0)
        plsc.subcore_barrier()
        pltpu.sync_copy(acc.at[pl.ds(s * ROWS_PT, ROWS_PT)],
                        out_hbm.at[c, pl.ds(s * ROWS_PT, ROWS_PT)])

    return agg


_sc_agg_h1h = _make_sc_agg_staged(H1H)


_sc_agg_h2 = _make_sc_agg_staged(H2)


@functools.partial(
    pl.kernel,
    out_type=jax.ShapeDtypeStruct((E_LP_PAD, H2), jnp.float32),
    mesh=_sc_mesh(),
    compiler_params=pltpu.CompilerParams(use_tc_tiling_on_sc=False),
    scratch_types=[
        pltpu.VMEM((KC_LP, CHUNK), jnp.int32),
        pltpu.VMEM((KC_LP, CHUNK), jnp.int32),
        pltpu.VMEM((CHUNK, H2), jnp.float32),
        pltpu.VMEM((CHUNK, H2), jnp.float32),
        pltpu.VMEM((CHUNK, H2), jnp.float32),
        pltpu.VMEM_SHARED((N_NODES, H2), jnp.float32),
        pltpu.SemaphoreType.DMA,
        pltpu.SemaphoreType.DMA,
    ],
)
def _sc_lp_sqdist(emb_hbm, a_hbm, b_hbm, out_hbm,
                  a_v, b_v, bufa, bufb, bufd, emb_s, sema, semb):
    c = lax.axis_index("c")
    s = lax.axis_index("s")
    w = s * NC + c
    base = w * LP_PW_PAD
    pltpu.sync_copy(emb_hbm.at[pl.ds(s * NROWS_STAGE, NROWS_STAGE)],
                    emb_s.at[pl.ds(s * NROWS_STAGE, NROWS_STAGE)])
    pltpu.sync_copy(a_hbm.at[w], a_v)
    pltpu.sync_copy(b_hbm.at[w], b_v)
    plsc.subcore_barrier()

    def body(j, carry):
        cpa = pltpu.async_copy(emb_s.at[a_v.at[j]], bufa, sema)
        cpb = pltpu.async_copy(emb_s.at[b_v.at[j]], bufb, semb)
        cpa.wait()
        cpb.wait()

        def row(r, cc):
            dv = bufa[r] - bufb[r]
            bufd[r] = dv * dv
            return cc

        lax.fori_loop(0, CHUNK, row, 0)
        pltpu.sync_copy(bufd, out_hbm.at[pl.ds(base + j * CHUNK, CHUNK)])
        return carry

    lax.fori_loop(0, KC_LP, body, 0)


def _deg_dinv(parts):
    deg = parts[0, :, 0] + parts[1, :, 0] + 1.0
    return lax.rsqrt(jnp.maximum(deg, 1.0))


def _tc_encode1_body(degp_ref, x_ref, w1_ref, oa_ref, ob_ref):
    dinv = _deg_dinv(degp_ref[...])
    hw = jnp.dot(x_ref[...], w1_ref[...], preferred_element_type=jnp.float32)
    g = hw * dinv[:, None]
    oa_ref[...] = g[:, :H1H]
    ob_ref[...] = g[:, H1H:]


def _tc_encode2_body(degp_ref, accpa_ref, accpb_ref, g1a_ref, g1b_ref,
                     w2_ref, b1_ref, o_ref):
    dinv = _deg_dinv(degp_ref[...])
    a = jnp.concatenate(
        [accpa_ref[0] + accpa_ref[1] + g1a_ref[...],
         accpb_ref[0] + accpb_ref[1] + g1b_ref[...]], axis=-1)
    h = jnp.maximum(a * dinv[:, None] + b1_ref[...], 0.0)
    o_ref[...] = jnp.dot(h, w2_ref[...],
                         preferred_element_type=jnp.float32) * dinv[:, None]


def _tc_emb_body(degp_ref, accp_ref, g2_ref, b2_ref, o_ref):
    dinv = _deg_dinv(degp_ref[...])
    a = accp_ref[0] + accp_ref[1] + g2_ref[...]
    emb = jnp.maximum(a * dinv[:, None] + b2_ref[...], 0.0)
    nrm = jnp.sqrt(jnp.sum(emb * emb, axis=1, keepdims=True))
    scale = jnp.where(nrm > 1.0, 1.0 / (nrm + 1e-7), 1.0)
    o_ref[...] = emb * scale


def _tc_decode_body(sq_ref, pi_ref, wa_ref, wb_ref, bl1_ref, wl_ref,
                    bl_ref, o_ref):
    z = (jnp.dot(sq_ref[...], wa_ref[...], preferred_element_type=jnp.float32)
         + jnp.dot(pi_ref[...], wb_ref[...], preferred_element_type=jnp.float32)
         + bl1_ref[...])
    z = jnp.where(z > 0, z, 0.2 * z)
    sv = jnp.dot(z, wl_ref[...], preferred_element_type=jnp.float32)[:, 0]
    s = jnp.clip(jnp.abs(sv + bl_ref[0, 0]), 0.0, 40.0)
    o_ref[...] = (1.0 / (jnp.exp(s - 2.0) + 1.0))[:, None]


def kernel(x, edge_index, lp_edges, PI, W1, b1, W2, b2, Wl1, bl1, Wl, bl):
    f32 = jnp.float32
    src = edge_index[0]
    dst = edge_index[1]
    pad = EDGE_PW_PAD - EDGE_PW
    srcm = jnp.pad(src.reshape(NW, EDGE_PW), ((0, 0), (0, pad))
                   ).reshape(NW, KC_EDGE, CHUNK)
    dstm = jnp.pad(dst.reshape(NW, EDGE_PW), ((0, 0), (0, pad)),
                   constant_values=N_NODES).reshape(NW, KC_EDGE, CHUNK)
    lpa = jnp.pad(lp_edges[:, 0], (0, E_LP_PAD - E_LP)
                  ).reshape(NW, KC_LP, CHUNK)
    lpb = jnp.pad(lp_edges[:, 1], (0, E_LP_PAD - E_LP)
                  ).reshape(NW, KC_LP, CHUNK)

    ones_blk = jnp.ones((CHUNK, H2), f32)
    zeros_h2 = jnp.zeros((ROWS_PT, H2), f32)
    zeros_h1h = jnp.zeros((ROWS_PT, H1H), f32)

    W1p = jnp.pad(W1, ((0, 0), (0, H1P - H1)))
    W2p = jnp.pad(W2, ((0, H1P - H1), (0, 0)))
    b1p = jnp.pad(b1, (0, H1P - H1)).reshape(1, H1P)

    deg_parts = _sc_degree(dstm, ones_blk, zeros_h2)
    degp = deg_parts[:, :N_NODES, :]

    g1a, g1b = pl.pallas_call(
        _tc_encode1_body,
        grid=(N_NODES // ROW_BLK,),
        in_specs=[
            pl.BlockSpec((2, ROW_BLK, H2), lambda i: (0, i, 0)),
            pl.BlockSpec((ROW_BLK, F_IN), lambda i: (i, 0)),
            pl.BlockSpec((F_IN, H1P), lambda i: (0, 0)),
        ],
        out_specs=[pl.BlockSpec((ROW_BLK, H1H), lambda i: (i, 0)),
                   pl.BlockSpec((ROW_BLK, H1H), lambda i: (i, 0))],
        out_shape=[jax.ShapeDtypeStruct((N_NODES, H1H), f32),
                   jax.ShapeDtypeStruct((N_NODES, H1H), f32)],
    )(degp, x, W1p)

    acc1a = _sc_agg_h1h(g1a, srcm, dstm, zeros_h1h)
    acc1b = _sc_agg_h1h(g1b, srcm, dstm, zeros_h1h)

    g2 = pl.pallas_call(
        _tc_encode2_body,
        grid=(N_NODES // ROW_BLK,),
        in_specs=[
            pl.BlockSpec((2, ROW_BLK, H2), lambda i: (0, i, 0)),
            pl.BlockSpec((2, ROW_BLK, H1H), lambda i: (0, i, 0)),
            pl.BlockSpec((2, ROW_BLK, H1H), lambda i: (0, i, 0)),
            pl.BlockSpec((ROW_BLK, H1H), lambda i: (i, 0)),
            pl.BlockSpec((ROW_BLK, H1H), lambda i: (i, 0)),
            pl.BlockSpec((H1P, H2), lambda i: (0, 0)),
            pl.BlockSpec((1, H1P), lambda i: (0, 0)),
        ],
        out_specs=pl.BlockSpec((ROW_BLK, H2), lambda i: (i, 0)),
        out_shape=jax.ShapeDtypeStruct((N_NODES, H2), f32),
    )(degp, acc1a[:, :N_NODES, :], acc1b[:, :N_NODES, :], g1a, g1b,
      W2p, b1p)

    acc2 = _sc_agg_h2(g2, srcm, dstm, zeros_h2)

    emb = pl.pallas_call(
        _tc_emb_body,
        grid=(N_NODES // ROW_BLK,),
        in_specs=[
            pl.BlockSpec((2, ROW_BLK, H2), lambda i: (0, i, 0)),
            pl.BlockSpec((2, ROW_BLK, H2), lambda i: (0, i, 0)),
            pl.BlockSpec((ROW_BLK, H2), lambda i: (i, 0)),
            pl.BlockSpec((1, H2), lambda i: (0, 0)),
        ],
        out_specs=pl.BlockSpec((ROW_BLK, H2), lambda i: (i, 0)),
        out_shape=jax.ShapeDtypeStruct((N_NODES, H2), f32),
    )(degp, acc2[:, :N_NODES, :], g2, b2.reshape(1, H2))

    sq = _sc_lp_sqdist(emb, lpa, lpb)

    prob = pl.pallas_call(
        _tc_decode_body,
        grid=(E_LP // LP_BLK,),
        in_specs=[
            pl.BlockSpec((LP_BLK, H2), lambda i: (i, 0)),
            pl.BlockSpec((LP_BLK, PI_D), lambda i: (i, 0)),
            pl.BlockSpec((H2, PI_D), lambda i: (0, 0)),
            pl.BlockSpec((PI_D, PI_D), lambda i: (0, 0)),
            pl.BlockSpec((1, PI_D), lambda i: (0, 0)),
            pl.BlockSpec((PI_D, 1), lambda i: (0, 0)),
            pl.BlockSpec((1, 1), lambda i: (0, 0)),
        ],
        out_specs=pl.BlockSpec((LP_BLK, 1), lambda i: (i, 0)),
        out_shape=jax.ShapeDtypeStruct((E_LP, 1), f32),
    )(sq[:E_LP], PI, Wl1[:H2], Wl1[H2:], bl1.reshape(1, PI_D), Wl,
      bl.reshape(1, 1))
    return prob.reshape(-1)
